# baseline (speedup 1.0000x reference)
import functools

import jax
import jax.numpy as jnp
from jax import lax
from jax.experimental import pallas as pl
from jax.experimental.pallas import tpu as pltpu
from jax.experimental.pallas import tpu_sc as plsc

_B = 16384
_D = 32
_NE = 1000000
_NC = 2
_NS = 16
_NW = _NC * _NS
_BPW = _B // _NW
_CHUNK = 128
_NCHUNK = _BPW // _CHUNK
_L = 16
_GPC = _CHUNK // _L
_NR = 1000

_PANEL = 16384
_GRID = -(-_NE // _PANEL)
_ROWS_PER_PANEL = _PANEL // 4
_NE4 = _GRID * _ROWS_PER_PANEL
_MPP = _PANEL // 128
_MLOW = _MPP // 4


def _relayout_body(ent_t_ref, out_ref):
    x = ent_t_ref[...].astype(jnp.bfloat16)
    eye = jnp.eye(128, dtype=jnp.bfloat16)

    def t(m):
        return lax.dot_general(
            eye, x[:, m * 128:(m + 1) * 128],
            (((1,), (1,)), ((), ())),
            preferred_element_type=jnp.float32,
        )

    for m_low in range(_MLOW):
        tile = jnp.concatenate(
            [t(k2 * _MLOW + m_low) for k2 in range(4)], axis=1)
        out_ref[pl.ds(m_low * 128, 128), :] = tile


_relayout_tc = pl.pallas_call(
    _relayout_body,
    grid=(_GRID,),
    in_specs=[pl.BlockSpec((_D, _PANEL), lambda g: (0, g))],
    out_specs=pl.BlockSpec((_ROWS_PER_PANEL, 128), lambda g: (g, 0)),
    out_shape=jax.ShapeDtypeStruct((_NE4, 128), jnp.float32),
)


def _rsqrt(x):
    i = plsc.bitcast(x, jnp.int32)
    i = jnp.int32(0x5F3759DF) - lax.shift_right_logical(i, 1)
    y = plsc.bitcast(i, jnp.float32)
    for _ in range(3):
        y = y * (1.5 - 0.5 * x * y * y)
    return y


@functools.partial(
    pl.kernel,
    out_type=jax.ShapeDtypeStruct((_B,), jnp.float32),
    mesh=plsc.VectorSubcoreMesh(core_axis_name="c", subcore_axis_name="s"),
    compiler_params=pltpu.CompilerParams(
        needs_layout_passes=False, disable_bounds_checks=True),
    scratch_types=[
        pltpu.VMEM((_NCHUNK, _CHUNK), jnp.int32),
        pltpu.VMEM((_NCHUNK, _CHUNK), jnp.int32),
        pltpu.VMEM((_NCHUNK, _CHUNK), jnp.int32),
        pltpu.VMEM((_NCHUNK, _CHUNK), jnp.int32),
        pltpu.VMEM((_NCHUNK, _CHUNK), jnp.int32),
        pltpu.VMEM((_CHUNK, _CHUNK), jnp.float32),
        pltpu.VMEM((_CHUNK, _CHUNK), jnp.float32),
        pltpu.VMEM((_CHUNK, _CHUNK), jnp.float32),
        pltpu.VMEM((_CHUNK, _CHUNK), jnp.float32),
        pltpu.VMEM((_D, _NR), jnp.float32),
        pltpu.VMEM((_BPW,), jnp.float32),
        pltpu.SemaphoreType.DMA,
        pltpu.SemaphoreType.DMA,
    ],
)
def _transe_sc(e1_hbm, rel_hbm, e2_hbm, ent4_hbm, reltab_hbm, out_hbm,
               idx1_v, idxr_v, idx2_v, sup1_v, sup2_v,
               head0_v, tail0_v, head1_v, tail1_v, reltab_v, dist_v,
               sem0, sem1):
    wid = lax.axis_index("s") * _NC + lax.axis_index("c")

    pltpu.sync_copy(e1_hbm.at[wid], idx1_v)
    pltpu.sync_copy(rel_hbm.at[wid], idxr_v)
    pltpu.sync_copy(e2_hbm.at[wid], idx2_v)
    pltpu.sync_copy(reltab_hbm, reltab_v)

    def superrow(e):
        return (lax.shift_left(lax.shift_right_logical(e, 14), 12)
                | (e & 4095))

    def shift(i, carry):
        j = lax.shift_right_logical(i, 3)
        base = (i - j * 8) * _L
        sl = pl.ds(base, _L)
        sup1_v[j, sl] = superrow(idx1_v[j, sl])
        sup2_v[j, sl] = superrow(idx2_v[j, sl])
        return carry

    lax.fori_loop(0, _NCHUNK * _GPC, shift, 0)

    bufs = ((head0_v, tail0_v, sem0), (head1_v, tail1_v, sem1))

    def fire(j):
        hv, tv, sm = bufs[j & 1]
        return [
            pltpu.async_copy(ent4_hbm.at[sup1_v.at[j]], hv, sm),
            pltpu.async_copy(ent4_hbm.at[sup2_v.at[j]], tv, sm),
        ]

    pending = fire(0)
    for j in range(_NCHUNK):
        nxt = fire(j + 1) if j + 1 < _NCHUNK else []
        for cp in pending:
            cp.wait()
        pending = nxt
        head_v, tail_v, _ = bufs[j & 1]

        def group(g, carry):
            base = g * _L
            sl = pl.ds(base, _L)
            rows = base + lax.iota(jnp.int32, _L)
            iv1 = idx1_v[j, sl]
            iv2 = idx2_v[j, sl]
            ivr = idxr_v[j, sl]
            c1 = (lax.shift_right_logical(iv1, 12) & 3) * _D
            c2 = (lax.shift_right_logical(iv2, 12) & 3) * _D
            zero = jnp.zeros((_L,), jnp.float32)
            ss_h, ss_r, ss_t = zero, zero, zero
            hr, ht, rt = zero, zero, zero
            for d in range(_D):
                h = plsc.load_gather(head_v, [rows, c1 + d])
                t = plsc.load_gather(tail_v, [rows, c2 + d])
                r = plsc.load_gather(reltab_v, [jnp.full((_L,), d, jnp.int32), ivr])
                ss_h += h * h
                ss_r += r * r
                ss_t += t * t
                hr += h * r
                ht += h * t
                rt += r * t
            inv_h = _rsqrt(jnp.maximum(ss_h, 1e-24))
            inv_t = _rsqrt(jnp.maximum(ss_t, 1e-24))
            d2 = ss_r + 2.0 + 2.0 * (inv_h * hr - inv_t * rt - inv_h * inv_t * ht)
            d2 = jnp.maximum(d2, 0.0)
            dist_v[pl.ds((j * _GPC + g) * _L, _L)] = d2 * _rsqrt(jnp.maximum(d2, 1e-30))
            return carry

        lax.fori_loop(0, _GPC, group, 0)

    pltpu.sync_copy(dist_v, out_hbm.at[pl.ds(wid * _BPW, _BPW)])


def kernel(e1_idx, rel_idx, e2_idx, emb_ent, emb_rel):
    e1 = e1_idx.astype(jnp.int32).reshape(_NW, _NCHUNK, _CHUNK)
    rr = rel_idx.astype(jnp.int32).reshape(_NW, _NCHUNK, _CHUNK)
    e2 = e2_idx.astype(jnp.int32).reshape(_NW, _NCHUNK, _CHUNK)
    ent4 = _relayout_tc(emb_ent.T)
    return _transe_sc(e1, rr, e2, ent4, emb_rel.T)


# speedup vs baseline: 1.1078x; 1.1078x over previous
import functools

import jax
import jax.numpy as jnp
from jax import lax
from jax.experimental import pallas as pl
from jax.experimental.pallas import tpu as pltpu
from jax.experimental.pallas import tpu_sc as plsc

_B = 16384
_D = 32
_NE = 1000000
_NC = 2
_NS = 16
_NW = _NC * _NS
_BPW = _B // _NW
_CHUNK = 128
_NCHUNK = _BPW // _CHUNK
_L = 16
_GPC = _CHUNK // _L
_NR = 1000

_PANEL = 32768
_GRID = -(-_NE // _PANEL)
_ROWS_PER_PANEL = _PANEL // 4
_NE4 = _GRID * _ROWS_PER_PANEL
_MPP = _PANEL // 128
_MLOW = _MPP // 4


def _relayout_body(ent_t_ref, out_ref):
    x = ent_t_ref[...].astype(jnp.bfloat16)
    eye = jnp.eye(128, dtype=jnp.bfloat16)

    def t(m):
        return lax.dot_general(
            eye, x[:, m * 128:(m + 1) * 128],
            (((1,), (1,)), ((), ())),
            preferred_element_type=jnp.float32,
        )

    for m_low in range(_MLOW):
        tile = jnp.concatenate(
            [t(k2 * _MLOW + m_low) for k2 in range(4)], axis=1)
        out_ref[pl.ds(m_low * 128, 128), :] = tile


_relayout_tc = pl.pallas_call(
    _relayout_body,
    grid=(_GRID,),
    in_specs=[pl.BlockSpec((_D, _PANEL), lambda g: (0, g))],
    out_specs=pl.BlockSpec((_ROWS_PER_PANEL, 128), lambda g: (g, 0)),
    out_shape=jax.ShapeDtypeStruct((_NE4, 128), jnp.float32),
)


def _rsqrt(x):
    i = plsc.bitcast(x, jnp.int32)
    i = jnp.int32(0x5F3759DF) - lax.shift_right_logical(i, 1)
    y = plsc.bitcast(i, jnp.float32)
    for _ in range(3):
        y = y * (1.5 - 0.5 * x * y * y)
    return y


@functools.partial(
    pl.kernel,
    out_type=jax.ShapeDtypeStruct((_B,), jnp.float32),
    mesh=plsc.VectorSubcoreMesh(core_axis_name="c", subcore_axis_name="s"),
    compiler_params=pltpu.CompilerParams(
        needs_layout_passes=False, disable_bounds_checks=True),
    scratch_types=[
        pltpu.VMEM((_NCHUNK, _CHUNK), jnp.int32),
        pltpu.VMEM((_NCHUNK, _CHUNK), jnp.int32),
        pltpu.VMEM((_NCHUNK, _CHUNK), jnp.int32),
        pltpu.VMEM((_NCHUNK, _CHUNK), jnp.int32),
        pltpu.VMEM((_NCHUNK, _CHUNK), jnp.int32),
        pltpu.VMEM((_CHUNK, _CHUNK), jnp.float32),
        pltpu.VMEM((_CHUNK, _CHUNK), jnp.float32),
        pltpu.VMEM((_CHUNK, _CHUNK), jnp.float32),
        pltpu.VMEM((_CHUNK, _CHUNK), jnp.float32),
        pltpu.VMEM((_D, _NR), jnp.float32),
        pltpu.VMEM((_BPW,), jnp.float32),
        pltpu.SemaphoreType.DMA,
        pltpu.SemaphoreType.DMA,
    ],
)
def _transe_sc(e1_hbm, rel_hbm, e2_hbm, ent4_hbm, reltab_hbm, out_hbm,
               idx1_v, idxr_v, idx2_v, sup1_v, sup2_v,
               head0_v, tail0_v, head1_v, tail1_v, reltab_v, dist_v,
               sem0, sem1):
    wid = lax.axis_index("s") * _NC + lax.axis_index("c")

    pltpu.sync_copy(e1_hbm.at[wid], idx1_v)
    pltpu.sync_copy(rel_hbm.at[wid], idxr_v)
    pltpu.sync_copy(e2_hbm.at[wid], idx2_v)
    pltpu.sync_copy(reltab_hbm, reltab_v)

    def superrow(e):
        return (lax.shift_left(lax.shift_right_logical(e, 15), 13)
                | (e & 8191))

    def shift(i, carry):
        j = lax.shift_right_logical(i, 3)
        base = (i - j * 8) * _L
        sl = pl.ds(base, _L)
        sup1_v[j, sl] = superrow(idx1_v[j, sl])
        sup2_v[j, sl] = superrow(idx2_v[j, sl])
        return carry

    lax.fori_loop(0, _NCHUNK * _GPC, shift, 0)

    bufs = ((head0_v, tail0_v, sem0), (head1_v, tail1_v, sem1))

    def fire(j):
        hv, tv, sm = bufs[j & 1]
        return [
            pltpu.async_copy(ent4_hbm.at[sup1_v.at[j]], hv, sm),
            pltpu.async_copy(ent4_hbm.at[sup2_v.at[j]], tv, sm),
        ]

    pending = fire(0)
    for j in range(_NCHUNK):
        nxt = fire(j + 1) if j + 1 < _NCHUNK else []
        for cp in pending:
            cp.wait()
        pending = nxt
        head_v, tail_v, _ = bufs[j & 1]

        def group(g, carry):
            base = g * _L
            sl = pl.ds(base, _L)
            rows = base + lax.iota(jnp.int32, _L)
            iv1 = idx1_v[j, sl]
            iv2 = idx2_v[j, sl]
            ivr = idxr_v[j, sl]
            c1 = (lax.shift_right_logical(iv1, 13) & 3) * _D
            c2 = (lax.shift_right_logical(iv2, 13) & 3) * _D
            zero = jnp.zeros((_L,), jnp.float32)
            ss_h, ss_r, ss_t = zero, zero, zero
            hr, ht, rt = zero, zero, zero
            for d in range(_D):
                h = plsc.load_gather(head_v, [rows, c1 + d])
                t = plsc.load_gather(tail_v, [rows, c2 + d])
                r = plsc.load_gather(reltab_v, [jnp.full((_L,), d, jnp.int32), ivr])
                ss_h += h * h
                ss_r += r * r
                ss_t += t * t
                hr += h * r
                ht += h * t
                rt += r * t
            inv_h = _rsqrt(jnp.maximum(ss_h, 1e-24))
            inv_t = _rsqrt(jnp.maximum(ss_t, 1e-24))
            d2 = ss_r + 2.0 + 2.0 * (inv_h * hr - inv_t * rt - inv_h * inv_t * ht)
            d2 = jnp.maximum(d2, 0.0)
            dist_v[pl.ds((j * _GPC + g) * _L, _L)] = d2 * _rsqrt(jnp.maximum(d2, 1e-30))
            return carry

        lax.fori_loop(0, _GPC, group, 0, unroll=2)

    pltpu.sync_copy(dist_v, out_hbm.at[pl.ds(wid * _BPW, _BPW)])


def kernel(e1_idx, rel_idx, e2_idx, emb_ent, emb_rel):
    e1 = e1_idx.astype(jnp.int32).reshape(_NW, _NCHUNK, _CHUNK)
    rr = rel_idx.astype(jnp.int32).reshape(_NW, _NCHUNK, _CHUNK)
    e2 = e2_idx.astype(jnp.int32).reshape(_NW, _NCHUNK, _CHUNK)
    ent4 = _relayout_tc(emb_ent.T)
    return _transe_sc(e1, rr, e2, ent4, emb_rel.T)


# speedup vs baseline: 1.1128x; 1.0045x over previous
import functools

import jax
import jax.numpy as jnp
from jax import lax
from jax.experimental import pallas as pl
from jax.experimental.pallas import tpu as pltpu
from jax.experimental.pallas import tpu_sc as plsc

_B = 16384
_D = 32
_NE = 1000000
_NC = 2
_NS = 16
_NW = _NC * _NS
_BPW = _B // _NW
_CHUNK = 128
_NCHUNK = _BPW // _CHUNK
_L = 16
_GPC = _CHUNK // _L
_NR = 1000

_PANEL = 65536
_GRID = -(-_NE // _PANEL)
_ROWS_PER_PANEL = _PANEL // 4
_NE4 = _GRID * _ROWS_PER_PANEL
_MPP = _PANEL // 128
_MLOW = _MPP // 4


def _relayout_body(ent_t_ref, out_ref):
    x = ent_t_ref[...].astype(jnp.bfloat16)
    eye = jnp.eye(128, dtype=jnp.bfloat16)

    def t(m):
        return lax.dot_general(
            eye, x[:, m * 128:(m + 1) * 128],
            (((1,), (1,)), ((), ())),
            preferred_element_type=jnp.float32,
        )

    for m_low in range(_MLOW):
        tile = jnp.concatenate(
            [t(k2 * _MLOW + m_low) for k2 in range(4)], axis=1)
        out_ref[pl.ds(m_low * 128, 128), :] = tile


_relayout_tc = pl.pallas_call(
    _relayout_body,
    grid=(_GRID,),
    in_specs=[pl.BlockSpec((_D, _PANEL), lambda g: (0, g))],
    out_specs=pl.BlockSpec((_ROWS_PER_PANEL, 128), lambda g: (g, 0)),
    out_shape=jax.ShapeDtypeStruct((_NE4, 128), jnp.float32),
)


def _rsqrt(x):
    i = plsc.bitcast(x, jnp.int32)
    i = jnp.int32(0x5F3759DF) - lax.shift_right_logical(i, 1)
    y = plsc.bitcast(i, jnp.float32)
    for _ in range(3):
        y = y * (1.5 - 0.5 * x * y * y)
    return y


@functools.partial(
    pl.kernel,
    out_type=jax.ShapeDtypeStruct((_B,), jnp.float32),
    mesh=plsc.VectorSubcoreMesh(core_axis_name="c", subcore_axis_name="s"),
    compiler_params=pltpu.CompilerParams(
        needs_layout_passes=False, disable_bounds_checks=True),
    scratch_types=[
        pltpu.VMEM((_NCHUNK, _CHUNK), jnp.int32),
        pltpu.VMEM((_NCHUNK, _CHUNK), jnp.int32),
        pltpu.VMEM((_NCHUNK, _CHUNK), jnp.int32),
        pltpu.VMEM((_NCHUNK, _CHUNK), jnp.int32),
        pltpu.VMEM((_NCHUNK, _CHUNK), jnp.int32),
        pltpu.VMEM((_CHUNK, _CHUNK), jnp.float32),
        pltpu.VMEM((_CHUNK, _CHUNK), jnp.float32),
        pltpu.VMEM((_CHUNK, _CHUNK), jnp.float32),
        pltpu.VMEM((_CHUNK, _CHUNK), jnp.float32),
        pltpu.VMEM((_D, _NR), jnp.float32),
        pltpu.VMEM((_BPW,), jnp.float32),
        pltpu.SemaphoreType.DMA,
        pltpu.SemaphoreType.DMA,
    ],
)
def _transe_sc(e1_hbm, rel_hbm, e2_hbm, ent4_hbm, reltab_hbm, out_hbm,
               idx1_v, idxr_v, idx2_v, sup1_v, sup2_v,
               head0_v, tail0_v, head1_v, tail1_v, reltab_v, dist_v,
               sem0, sem1):
    wid = lax.axis_index("s") * _NC + lax.axis_index("c")

    pltpu.sync_copy(e1_hbm.at[wid], idx1_v)
    pltpu.sync_copy(rel_hbm.at[wid], idxr_v)
    pltpu.sync_copy(e2_hbm.at[wid], idx2_v)
    pltpu.sync_copy(reltab_hbm, reltab_v)

    def superrow(e):
        return (lax.shift_left(lax.shift_right_logical(e, 16), 14)
                | (e & 16383))

    def shift(i, carry):
        j = lax.shift_right_logical(i, 3)
        base = (i - j * 8) * _L
        sl = pl.ds(base, _L)
        sup1_v[j, sl] = superrow(idx1_v[j, sl])
        sup2_v[j, sl] = superrow(idx2_v[j, sl])
        return carry

    lax.fori_loop(0, _NCHUNK * _GPC, shift, 0)

    bufs = ((head0_v, tail0_v, sem0), (head1_v, tail1_v, sem1))

    def fire(j):
        hv, tv, sm = bufs[j & 1]
        return [
            pltpu.async_copy(ent4_hbm.at[sup1_v.at[j]], hv, sm),
            pltpu.async_copy(ent4_hbm.at[sup2_v.at[j]], tv, sm),
        ]

    pending = fire(0)
    for j in range(_NCHUNK):
        nxt = fire(j + 1) if j + 1 < _NCHUNK else []
        for cp in pending:
            cp.wait()
        pending = nxt
        head_v, tail_v, _ = bufs[j & 1]

        def group(g, carry):
            base = g * _L
            sl = pl.ds(base, _L)
            rows = base + lax.iota(jnp.int32, _L)
            iv1 = idx1_v[j, sl]
            iv2 = idx2_v[j, sl]
            ivr = idxr_v[j, sl]
            c1 = (lax.shift_right_logical(iv1, 14) & 3) * _D
            c2 = (lax.shift_right_logical(iv2, 14) & 3) * _D
            zero = jnp.zeros((_L,), jnp.float32)
            ss_h, ss_r, ss_t = zero, zero, zero
            hr, ht, rt = zero, zero, zero
            for d in range(_D):
                h = plsc.load_gather(head_v, [rows, c1 + d])
                t = plsc.load_gather(tail_v, [rows, c2 + d])
                r = plsc.load_gather(reltab_v, [jnp.full((_L,), d, jnp.int32), ivr])
                ss_h += h * h
                ss_r += r * r
                ss_t += t * t
                hr += h * r
                ht += h * t
                rt += r * t
            inv_h = _rsqrt(jnp.maximum(ss_h, 1e-24))
            inv_t = _rsqrt(jnp.maximum(ss_t, 1e-24))
            d2 = ss_r + 2.0 + 2.0 * (inv_h * hr - inv_t * rt - inv_h * inv_t * ht)
            d2 = jnp.maximum(d2, 0.0)
            dist_v[pl.ds((j * _GPC + g) * _L, _L)] = d2 * _rsqrt(jnp.maximum(d2, 1e-30))
            return carry

        lax.fori_loop(0, _GPC, group, 0, unroll=2)

    pltpu.sync_copy(dist_v, out_hbm.at[pl.ds(wid * _BPW, _BPW)])


def kernel(e1_idx, rel_idx, e2_idx, emb_ent, emb_rel):
    e1 = e1_idx.astype(jnp.int32).reshape(_NW, _NCHUNK, _CHUNK)
    rr = rel_idx.astype(jnp.int32).reshape(_NW, _NCHUNK, _CHUNK)
    e2 = e2_idx.astype(jnp.int32).reshape(_NW, _NCHUNK, _CHUNK)
    ent4 = _relayout_tc(emb_ent.T)
    return _transe_sc(e1, rr, e2, ent4, emb_rel.T)
